# baseline (speedup 1.0000x reference)
import functools
import math

import jax
import jax.numpy as jnp
from jax import lax
from jax.experimental import pallas as pl
from jax.experimental.pallas import tpu as pltpu
from jax.experimental.pallas import tpu_sc as plsc

S = 4096
B = 4
D = 1024
LANES = 16
SCALE = 1.0 / math.sqrt(D)

S_TC = 2560
S_SC = S - S_TC
BS_TC = 256

NC = 2
NS = 16
NW = NC * NS
S_PER_W = S_SC // NW
CH = 8
NSTEPS = S_PER_W // CH
NBUF = 3


def _sc_body(x_hbm, pos_hbm, out_hbm, xbuf, pbuf, xsem, psem, osem):
    wid = lax.axis_index("s") * NC + lax.axis_index("c")
    base = wid * S_PER_W

    def compute(slot):
        nvec = D // LANES

        @plsc.parallel_loop(0, CH * nvec, unroll=8)
        def _(j):
            i = j // nvec
            d = (j % nvec) * LANES
            pv = pbuf[slot, i, pl.ds(d, LANES)] * SCALE
            for b in range(B):
                plsc.addupdate(xbuf.at[slot, i, b, pl.ds(d, LANES)], pv)

    copies = {}

    def start_in(step):
        slot = step % NBUF
        s0 = S_TC + base + step * CH
        copies[("x", step)] = pltpu.async_copy(
            x_hbm.at[pl.ds(s0, CH)], xbuf.at[slot], xsem.at[slot])
        copies[("p", step)] = pltpu.async_copy(
            pos_hbm.at[pl.ds(s0, CH)], pbuf.at[slot], psem.at[slot])

    start_in(0)
    for step in range(NSTEPS):
        slot = step % NBUF
        if step + 1 < NSTEPS:
            if step + 1 >= NBUF:
                copies[("o", step + 1 - NBUF)].wait()
            start_in(step + 1)
        copies[("x", step)].wait()
        copies[("p", step)].wait()
        compute(slot)
        copies[("o", step)] = pltpu.async_copy(
            xbuf.at[slot], out_hbm.at[pl.ds(base + step * CH, CH)],
            osem.at[slot])
    for step in range(NSTEPS - NBUF + 1, NSTEPS):
        copies[("o", step - 1)].wait()
    copies[("o", NSTEPS - 1)].wait()


def _sc_part(x, pos_embedding):
    mesh = plsc.VectorSubcoreMesh(core_axis_name="c", subcore_axis_name="s")
    run = functools.partial(
        pl.kernel,
        mesh=mesh,
        out_type=jax.ShapeDtypeStruct((S_SC, B, D), jnp.float32),
        scratch_types=[
            pltpu.VMEM((NBUF, CH, B, D), jnp.float32),
            pltpu.VMEM((NBUF, CH, D), jnp.float32),
            pltpu.SemaphoreType.DMA((NBUF,)),
            pltpu.SemaphoreType.DMA((NBUF,)),
            pltpu.SemaphoreType.DMA((NBUF,)),
        ],
    )(_sc_body)
    return run(x, pos_embedding)


def _tc_body(x_ref, p_ref, o_ref):
    o_ref[...] = x_ref[...] + p_ref[...][:, None, :] * SCALE


def _tc_part(x, pos_embedding):
    return pl.pallas_call(
        _tc_body,
        grid=(S_TC // BS_TC,),
        in_specs=[
            pl.BlockSpec((BS_TC, B, D), lambda i: (i, 0, 0)),
            pl.BlockSpec((BS_TC, D), lambda i: (i, 0)),
        ],
        out_specs=pl.BlockSpec((BS_TC, B, D), lambda i: (i, 0, 0)),
        out_shape=jax.ShapeDtypeStruct((S_TC, B, D), jnp.float32),
    )(x, pos_embedding)


@jax.jit
def kernel(x, pos_embedding):
    tc_out = _tc_part(x, pos_embedding)
    sc_out = _sc_part(x, pos_embedding)
    return jnp.concatenate([tc_out, sc_out], axis=0)


# speedup vs baseline: 1.6750x; 1.6750x over previous
import functools
import math

import jax
import jax.numpy as jnp
from jax import lax
from jax.experimental import pallas as pl
from jax.experimental.pallas import tpu as pltpu
from jax.experimental.pallas import tpu_sc as plsc

S = 4096
B = 4
D = 1024
LANES = 16
SCALE = 1.0 / math.sqrt(D)

S_TC = 2560
S_SC = S - S_TC
BS_TC = 256

NC = 2
NS = 16
NW = NC * NS
S_PER_W = S_SC // NW
CH = 8
NSTEPS = S_PER_W // CH
NBUF = 3


def _sc_body(x_hbm, pos_hbm, out_hbm, xbuf, pbuf, xsem, psem, osem):
    wid = lax.axis_index("s") * NC + lax.axis_index("c")
    base = wid * S_PER_W

    def compute(slot):
        nvec = D // LANES

        @plsc.parallel_loop(0, CH * nvec, unroll=8)
        def _(j):
            i = j // nvec
            d = (j % nvec) * LANES
            pv = pbuf[slot, i, pl.ds(d, LANES)] * SCALE
            for b in range(B):
                plsc.addupdate(xbuf.at[slot, i, b, pl.ds(d, LANES)], pv)

    copies = {}

    def start_in(step):
        slot = step % NBUF
        s0 = S_TC + base + step * CH
        copies[("x", step)] = pltpu.async_copy(
            x_hbm.at[pl.ds(s0, CH)], xbuf.at[slot], xsem.at[slot])
        copies[("p", step)] = pltpu.async_copy(
            pos_hbm.at[pl.ds(s0, CH)], pbuf.at[slot], psem.at[slot])

    start_in(0)
    for step in range(NSTEPS):
        slot = step % NBUF
        if step + 1 < NSTEPS:
            if step + 1 >= NBUF:
                copies[("o", step + 1 - NBUF)].wait()
            start_in(step + 1)
        copies[("x", step)].wait()
        copies[("p", step)].wait()
        compute(slot)
        copies[("o", step)] = pltpu.async_copy(
            xbuf.at[slot], out_hbm.at[pl.ds(base + step * CH, CH)],
            osem.at[slot])
    for step in range(NSTEPS - NBUF + 1, NSTEPS):
        copies[("o", step - 1)].wait()
    copies[("o", NSTEPS - 1)].wait()


def _sc_part(x, pos_embedding):
    mesh = plsc.VectorSubcoreMesh(core_axis_name="c", subcore_axis_name="s")
    run = functools.partial(
        pl.kernel,
        mesh=mesh,
        out_type=jax.ShapeDtypeStruct((S_SC, B, D), jnp.float32),
        scratch_types=[
            pltpu.VMEM((NBUF, CH, B, D), jnp.float32),
            pltpu.VMEM((NBUF, CH, D), jnp.float32),
            pltpu.SemaphoreType.DMA((NBUF,)),
            pltpu.SemaphoreType.DMA((NBUF,)),
            pltpu.SemaphoreType.DMA((NBUF,)),
        ],
    )(_sc_body)
    return run(x, pos_embedding)


def _tc_body(x_ref, p_ref, o_ref):
    o_ref[...] = x_ref[...] + p_ref[...][:, None, :] * SCALE


def _tc_part(x, pos_embedding):
    return pl.pallas_call(
        _tc_body,
        grid=(S_TC // BS_TC,),
        in_specs=[
            pl.BlockSpec((BS_TC, B, D), lambda i: (i, 0, 0)),
            pl.BlockSpec((BS_TC, D), lambda i: (i, 0)),
        ],
        out_specs=pl.BlockSpec((BS_TC, B, D), lambda i: (i, 0, 0)),
        out_shape=jax.ShapeDtypeStruct((S_TC, B, D), jnp.float32),
    )(x, pos_embedding)


def _tc_full(x, pos_embedding):
    return pl.pallas_call(
        _tc_body,
        grid=(S_TC // BS_TC,),
        in_specs=[
            pl.BlockSpec((BS_TC, B, D), lambda i: (i, 0, 0)),
            pl.BlockSpec((BS_TC, D), lambda i: (i, 0)),
        ],
        out_specs=pl.BlockSpec((BS_TC, B, D), lambda i: (i, 0, 0)),
        out_shape=jax.ShapeDtypeStruct((S, B, D), jnp.float32),
    )(x, pos_embedding)


@jax.jit
def kernel(x, pos_embedding):
    tc_out = _tc_full(x, pos_embedding)
    sc_out = _sc_part(x, pos_embedding)
    return lax.dynamic_update_slice(tc_out, sc_out, (S_TC, 0, 0))


# speedup vs baseline: 1.7785x; 1.0618x over previous
import functools
import math

import jax
import jax.numpy as jnp
from jax import lax
from jax.experimental import pallas as pl
from jax.experimental.pallas import tpu as pltpu
from jax.experimental.pallas import tpu_sc as plsc

S = 4096
B = 4
D = 1024
LANES = 16
SCALE = 1.0 / math.sqrt(D)

NC = 2
NS = 16
NW = NC * NS
S_PER_W = S // NW
CH = 4
NSTEPS = S_PER_W // CH
NBUF = 6


def _body(x_hbm, pos_hbm, out_hbm, xbuf, pbuf,
          xsem, psem, osem):
    wid = lax.axis_index("s") * NC + lax.axis_index("c")
    base = wid * S_PER_W

    def compute(slot):
        nvec = D // LANES

        @plsc.parallel_loop(0, CH * nvec, unroll=8)
        def _(j):
            i = j // nvec
            d = (j % nvec) * LANES
            pv = pbuf[slot, i, pl.ds(d, LANES)] * SCALE
            for b in range(B):
                plsc.addupdate(xbuf.at[slot, i, b, pl.ds(d, LANES)], pv)

    copies = {}

    def start_in(step):
        slot = step % NBUF
        s0 = base + step * CH
        copies[("x", step)] = pltpu.async_copy(
            x_hbm.at[pl.ds(s0, CH)], xbuf.at[slot], xsem.at[slot])
        copies[("p", step)] = pltpu.async_copy(
            pos_hbm.at[pl.ds(s0, CH)], pbuf.at[slot], psem.at[slot])

    start_in(0)
    for step in range(NSTEPS):
        slot = step % NBUF
        if step + 1 < NSTEPS:
            if step + 1 >= NBUF:
                copies[("o", step + 1 - NBUF)].wait()
            start_in(step + 1)
        copies[("x", step)].wait()
        copies[("p", step)].wait()
        compute(slot)
        copies[("o", step)] = pltpu.async_copy(
            xbuf.at[slot], out_hbm.at[pl.ds(base + step * CH, CH)],
            osem.at[slot])
    for step in range(NSTEPS - NBUF + 1, NSTEPS):
        copies[("o", step - 1)].wait()
    copies[("o", NSTEPS - 1)].wait()


@jax.jit
def kernel(x, pos_embedding):
    mesh = plsc.VectorSubcoreMesh(core_axis_name="c", subcore_axis_name="s")
    run = functools.partial(
        pl.kernel,
        mesh=mesh,
        out_type=jax.ShapeDtypeStruct((S, B, D), jnp.float32),
        scratch_types=[
            pltpu.VMEM((NBUF, CH, B, D), jnp.float32),
            pltpu.VMEM((NBUF, CH, D), jnp.float32),
            pltpu.SemaphoreType.DMA((NBUF,)),
            pltpu.SemaphoreType.DMA((NBUF,)),
            pltpu.SemaphoreType.DMA((NBUF,)),
        ],
    )(_body)
    return run(x, pos_embedding)


# speedup vs baseline: 1.8949x; 1.0655x over previous
import functools
import math

import jax
import jax.numpy as jnp
from jax import lax
from jax.experimental import pallas as pl
from jax.experimental.pallas import tpu as pltpu
from jax.experimental.pallas import tpu_sc as plsc

S = 4096
B = 4
D = 1024
LANES = 16
SCALE = 1.0 / math.sqrt(D)

NC = 2
NS = 16
NW = NC * NS
S_PER_W = S // NW
CH = 8
NSTEPS = S_PER_W // CH
NBUF = 3


def _body(x_hbm, pos_hbm, out_hbm, xbuf, pbuf,
          xsem, psem, osem):
    wid = lax.axis_index("s") * NC + lax.axis_index("c")
    base = wid * S_PER_W

    def compute(slot):
        nvec = D // LANES

        @plsc.parallel_loop(0, CH * nvec, unroll=2)
        def _(j):
            i = j // nvec
            d = (j % nvec) * LANES
            pv = pbuf[slot, i, pl.ds(d, LANES)] * SCALE
            for b in range(B):
                plsc.addupdate(xbuf.at[slot, i, b, pl.ds(d, LANES)], pv)

    copies = {}

    def start_in(step):
        slot = step % NBUF
        s0 = base + step * CH
        copies[("x", step)] = pltpu.async_copy(
            x_hbm.at[pl.ds(s0, CH)], xbuf.at[slot], xsem.at[slot])
        copies[("p", step)] = pltpu.async_copy(
            pos_hbm.at[pl.ds(s0, CH)], pbuf.at[slot], psem.at[slot])

    start_in(0)
    for step in range(NSTEPS):
        slot = step % NBUF
        if step + 1 < NSTEPS:
            if step + 1 >= NBUF:
                copies[("o", step + 1 - NBUF)].wait()
            start_in(step + 1)
        copies[("x", step)].wait()
        copies[("p", step)].wait()
        compute(slot)
        copies[("o", step)] = pltpu.async_copy(
            xbuf.at[slot], out_hbm.at[pl.ds(base + step * CH, CH)],
            osem.at[slot])
    for step in range(NSTEPS - NBUF + 1, NSTEPS):
        copies[("o", step - 1)].wait()
    copies[("o", NSTEPS - 1)].wait()


@jax.jit
def kernel(x, pos_embedding):
    mesh = plsc.VectorSubcoreMesh(core_axis_name="c", subcore_axis_name="s")
    run = functools.partial(
        pl.kernel,
        mesh=mesh,
        out_type=jax.ShapeDtypeStruct((S, B, D), jnp.float32),
        scratch_types=[
            pltpu.VMEM((NBUF, CH, B, D), jnp.float32),
            pltpu.VMEM((NBUF, CH, D), jnp.float32),
            pltpu.SemaphoreType.DMA((NBUF,)),
            pltpu.SemaphoreType.DMA((NBUF,)),
            pltpu.SemaphoreType.DMA((NBUF,)),
        ],
    )(_body)
    return run(x, pos_embedding)


# speedup vs baseline: 1.8962x; 1.0007x over previous
import functools
import math

import jax
import jax.numpy as jnp
from jax import lax
from jax.experimental import pallas as pl
from jax.experimental.pallas import tpu as pltpu
from jax.experimental.pallas import tpu_sc as plsc

S = 4096
B = 4
D = 1024
LANES = 16
SCALE = 1.0 / math.sqrt(D)

NC = 2
NS = 16
NW = NC * NS
S_PER_W = S // NW
CH = 8
NSTEPS = S_PER_W // CH
NBUF = 3


def _body(x_hbm, pos_hbm, out_hbm, xbuf, pbuf,
          xsem, psem, osem):
    wid = lax.axis_index("s") * NC + lax.axis_index("c")
    base = wid * S_PER_W

    def compute(slot):
        nvec = D // LANES

        @plsc.parallel_loop(0, CH * nvec, unroll=4)
        def _(j):
            i = j // nvec
            d = (j % nvec) * LANES
            pv = pbuf[slot, i, pl.ds(d, LANES)] * SCALE
            for b in range(B):
                plsc.addupdate(xbuf.at[slot, i, b, pl.ds(d, LANES)], pv)

    copies = {}

    def start_in(step):
        slot = step % NBUF
        s0 = base + step * CH
        copies[("x", step)] = pltpu.async_copy(
            x_hbm.at[pl.ds(s0, CH)], xbuf.at[slot], xsem.at[slot])
        copies[("p", step)] = pltpu.async_copy(
            pos_hbm.at[pl.ds(s0, CH)], pbuf.at[slot], psem.at[slot])

    start_in(0)
    for step in range(NSTEPS):
        slot = step % NBUF
        if step + 1 < NSTEPS:
            if step + 1 >= NBUF:
                copies[("o", step + 1 - NBUF)].wait()
            start_in(step + 1)
        copies[("x", step)].wait()
        copies[("p", step)].wait()
        compute(slot)
        copies[("o", step)] = pltpu.async_copy(
            xbuf.at[slot], out_hbm.at[pl.ds(base + step * CH, CH)],
            osem.at[slot])
    for step in range(NSTEPS - NBUF + 1, NSTEPS):
        copies[("o", step - 1)].wait()
    copies[("o", NSTEPS - 1)].wait()


@jax.jit
def kernel(x, pos_embedding):
    mesh = plsc.VectorSubcoreMesh(core_axis_name="c", subcore_axis_name="s")
    run = functools.partial(
        pl.kernel,
        mesh=mesh,
        out_type=jax.ShapeDtypeStruct((S, B, D), jnp.float32),
        scratch_types=[
            pltpu.VMEM((NBUF, CH, B, D), jnp.float32),
            pltpu.VMEM((NBUF, CH, D), jnp.float32),
            pltpu.SemaphoreType.DMA((NBUF,)),
            pltpu.SemaphoreType.DMA((NBUF,)),
            pltpu.SemaphoreType.DMA((NBUF,)),
        ],
    )(_body)
    return run(x, pos_embedding)
